# baseline (speedup 1.0000x reference)
import jax
import jax.numpy as jnp
from jax.experimental import pallas as pl
from jax.experimental.pallas import tpu as pltpu

_BLK = 2048


def _body(ind_ref, x_ref, w_ref, b_ref, o_ref):
    x = x_ref[...]
    acc = jnp.dot(x, w_ref[...], preferred_element_type=jnp.float32)
    acc = acc + b_ref[...]
    mask = ind_ref[...] != 0
    o_ref[...] = jnp.where(mask, acc, 0.0)


def kernel(indicators, data, W, b):
    N, D = data.shape
    nb = N // _BLK
    ind2 = indicators.reshape(N, 1)
    b2 = b.reshape(1, D)
    return pl.pallas_call(
        _body,
        grid=(nb,),
        in_specs=[
            pl.BlockSpec((_BLK, 1), lambda i: (i, 0)),
            pl.BlockSpec((_BLK, D), lambda i: (i, 0)),
            pl.BlockSpec((D, D), lambda i: (0, 0)),
            pl.BlockSpec((1, D), lambda i: (0, 0)),
        ],
        out_specs=pl.BlockSpec((_BLK, D), lambda i: (i, 0)),
        out_shape=jax.ShapeDtypeStruct((N, D), jnp.float32),
        compiler_params=pltpu.CompilerParams(
            dimension_semantics=("arbitrary",),
        ),
    )(ind2, data, W, b2)


# speedup vs baseline: 1.1390x; 1.1390x over previous
import jax
import jax.numpy as jnp
from jax.experimental import pallas as pl
from jax.experimental.pallas import tpu as pltpu

_BLK = 2048


def _body(ind_ref, x_ref, w_ref, b_ref, o_ref):
    x = x_ref[...]
    acc = jnp.dot(x, w_ref[...], preferred_element_type=jnp.float32)
    acc = acc + b_ref[...]
    mask_row = (ind_ref[0] != 0).astype(jnp.float32)
    mask_col = jnp.transpose(mask_row)
    o_ref[...] = acc * mask_col


def kernel(indicators, data, W, b):
    N, D = data.shape
    nb = N // _BLK
    ind3 = indicators.reshape(nb, 1, _BLK)
    b2 = b.reshape(1, D)
    return pl.pallas_call(
        _body,
        grid=(nb,),
        in_specs=[
            pl.BlockSpec((1, 1, _BLK), lambda i: (i, 0, 0)),
            pl.BlockSpec((_BLK, D), lambda i: (i, 0)),
            pl.BlockSpec((D, D), lambda i: (0, 0)),
            pl.BlockSpec((1, D), lambda i: (0, 0)),
        ],
        out_specs=pl.BlockSpec((_BLK, D), lambda i: (i, 0)),
        out_shape=jax.ShapeDtypeStruct((N, D), jnp.float32),
        compiler_params=pltpu.CompilerParams(
            dimension_semantics=("arbitrary",),
        ),
    )(ind3, data, W, b2)
